# baseline (speedup 1.0000x reference)
import functools

import jax
import jax.numpy as jnp
from jax import lax
from jax.experimental import pallas as pl
from jax.experimental.pallas import tpu as pltpu
from jax.experimental.pallas import tpu_sc as plsc

B = 16384
RANK = 32
NW = 32
BPW = B // NW
NCH = BPW // 128
GPC = 128 // 16

_mesh = plsc.VectorSubcoreMesh(core_axis_name="c", subcore_axis_name="s")


@functools.partial(
    pl.kernel,
    mesh=_mesh,
    out_type=jax.ShapeDtypeStruct((B,), jnp.float32),
    scratch_types=[
        pltpu.VMEM((NCH, 128), jnp.int32),
        pltpu.VMEM((NCH, 128), jnp.int32),
        pltpu.VMEM((NCH, 128), jnp.int32),
        pltpu.VMEM((NCH, 128), jnp.int32),
        pltpu.VMEM((2, 128, 128), jnp.float32),
        pltpu.VMEM((2, 128, 128), jnp.float32),
        pltpu.VMEM((BPW,), jnp.float32),
        pltpu.SemaphoreType.DMA,
        pltpu.SemaphoreType.DMA,
    ],
    compiler_params=pltpu.CompilerParams(needs_layout_passes=False),
)
def _mf_kernel(user_hbm, anime_hbm, u_hbm, v_hbm, out_hbm,
               uidx, aidx, gu, gv, u_rows, v_rows, out_v, sem0, sem1):
    wid = lax.axis_index("s") * 2 + lax.axis_index("c")
    sems = [sem0, sem1]

    pltpu.sync_copy(user_hbm.at[pl.ds(wid * NCH, NCH)], uidx)
    pltpu.sync_copy(anime_hbm.at[pl.ds(wid * NCH, NCH)], aidx)

    for k in range(NCH):
        for g in range(GPC):
            s = pl.ds(g * 16, 16)
            gu[k, s] = lax.shift_right_logical(uidx[k, s], 2)
            gv[k, s] = lax.shift_right_logical(aidx[k, s], 2)

    def start_gather(c):
        buf = c % 2
        return (
            pltpu.async_copy(u_hbm.at[gu.at[c]], u_rows.at[buf], sems[buf]),
            pltpu.async_copy(v_hbm.at[gv.at[c]], v_rows.at[buf], sems[buf]),
        )

    lane = lax.iota(jnp.int32, 16)
    three = jnp.full((16,), 3, jnp.int32)

    def compute_chunk(c):
        buf = c % 2

        def group_body(g, carry):
            s = pl.ds(g * 16, 16)
            cbu = lax.shift_left(lax.bitwise_and(uidx[c, s], three), 5)
            cbv = lax.shift_left(lax.bitwise_and(aidx[c, s], three), 5)
            row = g * 16 + lane
            acc = jnp.zeros((16,), jnp.float32)
            for j in range(RANK):
                uu = plsc.load_gather(u_rows.at[buf], [row, cbu + j])
                vv = plsc.load_gather(v_rows.at[buf], [row, cbv + j])
                acc = acc + uu * vv
            out_v[pl.ds(c * 128 + g * 16, 16)] = acc
            return carry

        lax.fori_loop(0, GPC, group_body, 0)

    pending = start_gather(0)
    for c in range(NCH):
        nxt = start_gather(c + 1) if c + 1 < NCH else None
        for cp in pending:
            cp.wait()
        compute_chunk(c)
        pending = nxt

    pltpu.sync_copy(out_v, out_hbm.at[pl.ds(wid * BPW, BPW)])


def kernel(user, anime, U, V):
    user = user.astype(jnp.int32).reshape(NW * NCH, 128)
    anime = anime.astype(jnp.int32).reshape(NW * NCH, 128)
    u2 = U.reshape(-1, 128)
    v2 = V.reshape(-1, 128)
    return _mf_kernel(user, anime, u2, v2)
